# baseline (speedup 1.0000x reference)
import functools

import jax
import jax.numpy as jnp
from jax import lax
from jax.experimental import pallas as pl
from jax.experimental.pallas import tpu as pltpu
from jax.experimental.pallas import tpu_sc as plsc

DIM = 128
MARGIN = 4.0
BATCH = 4096
NC = 2
NS = 16
NW = NC * NS
L = 16
TRIPLES_PER_W = BATCH // NW
GROUPS = TRIPLES_PER_W // L


def _rsqrt(x):
    x = jnp.maximum(x, 1e-30)
    i = plsc.bitcast(x, jnp.int32)
    i = 0x5F3759DF - (i >> 1)
    y = plsc.bitcast(i, jnp.float32)
    for _ in range(3):
        y = y * (1.5 - 0.5 * x * y * y)
    return y


def _distance(bufs, lanes):
    hb, rb, tb, htb, rtb, ttb = bufs
    zeros = jnp.zeros((L,), jnp.float32)

    def body(d, c):
        dcol = jnp.broadcast_to(d, (L,)).astype(jnp.int32)
        h = plsc.load_gather(hb, [lanes, dcol])
        r = plsc.load_gather(rb, [lanes, dcol])
        t = plsc.load_gather(tb, [lanes, dcol])
        ht = plsc.load_gather(htb, [lanes, dcol])
        rt = plsc.load_gather(rtb, [lanes, dcol])
        tt = plsc.load_gather(ttb, [lanes, dcol])
        return (
            c[0] + h * ht,
            c[1] + t * tt,
            c[2] + h * r,
            c[3] + h * t,
            c[4] + r * t,
            c[5] + h * rt,
            c[6] + t * rt,
            c[7] + r * rt,
            c[8] + rt * rt,
            c[9] + r * r,
            c[10] + h * h,
            c[11] + t * t,
        )

    (sh, st, hr, ht_d, rt_d, hrt, trt, rrt, rtrt, rr, hh, tt2) = lax.fori_loop(
        0, DIM, body, (zeros,) * 12)

    nh2 = hh + 2.0 * sh * hrt + sh * sh * rtrt
    nt2 = tt2 + 2.0 * st * trt + st * st * rtrt
    hp_r = hr + sh * rrt
    hp_tp = ht_d + st * hrt + sh * trt + sh * st * rtrt
    r_tp = rt_d + st * rrt
    inh = _rsqrt(nh2)
    int_ = _rsqrt(nt2)
    inr = _rsqrt(rr)
    d2 = 3.0 + 2.0 * (hp_r * inh * inr - hp_tp * inh * int_ - r_tp * inr * int_)
    d2 = jnp.maximum(d2, 0.0)
    return d2 * _rsqrt(d2)


def _make_sc_kernel():
    mesh = plsc.VectorSubcoreMesh(core_axis_name="c", subcore_axis_name="s")

    @functools.partial(
        pl.kernel,
        mesh=mesh,
        compiler_params=pltpu.CompilerParams(needs_layout_passes=False),
        out_type=jax.ShapeDtypeStruct((NW, L), jnp.float32),
        scratch_types=(
            [pltpu.VMEM((6, TRIPLES_PER_W), jnp.int32)]
            + [pltpu.VMEM((L, DIM), jnp.float32) for _ in range(12)]
            + [pltpu.VMEM((L,), jnp.float32), pltpu.SemaphoreType.DMA]
        ),
    )
    def sc_kernel(ent_emb, rel_emb, ent_tr, rel_tr, idx_hbm, out_hbm,
                  idx_v, b0, b1, b2, b3, b4, b5, b6, b7, b8, b9, b10, b11,
                  acc_v, sem):
        wid = lax.axis_index("s") * NC + lax.axis_index("c")
        base = wid * TRIPLES_PER_W
        for j in range(6):
            pltpu.sync_copy(idx_hbm.at[j, pl.ds(base, TRIPLES_PER_W)],
                            idx_v.at[j])

        lanes = lax.iota(jnp.int32, L)
        pos_bufs = (b0, b1, b2, b3, b4, b5)
        neg_bufs = (b6, b7, b8, b9, b10, b11)
        plan = [(ent_emb, 0), (rel_emb, 1), (ent_emb, 2),
                (ent_tr, 0), (rel_tr, 1), (ent_tr, 2),
                (ent_emb, 3), (rel_emb, 4), (ent_emb, 5),
                (ent_tr, 3), (rel_tr, 4), (ent_tr, 5)]

        def group(g, acc):
            off = g * L
            copies = [
                pltpu.async_copy(tbl.at[idx_v.at[row, pl.ds(off, L)]], buf, sem)
                for buf, (tbl, row) in zip(pos_bufs + neg_bufs, plan)
            ]
            for c in copies:
                c.wait()
            pos = _distance(pos_bufs, lanes)
            neg = _distance(neg_bufs, lanes)
            return acc + jnp.maximum(pos - neg + MARGIN, 0.0)

        acc = lax.fori_loop(0, GROUPS, group, jnp.zeros((L,), jnp.float32))
        acc_v[...] = acc
        pltpu.sync_copy(acc_v, out_hbm.at[wid])

    return sc_kernel


_SC_KERNEL = _make_sc_kernel()


@jax.jit
def kernel(current_triples, corrupted_triples, ent_embedding, rel_embedding,
           ent_transfer, rel_transfer):
    idx = jnp.concatenate(
        [current_triples.T, corrupted_triples.T], axis=0).astype(jnp.int32)
    partials = _SC_KERNEL(ent_embedding, rel_embedding, ent_transfer,
                          rel_transfer, idx)
    return jnp.sum(partials) / BATCH


# speedup vs baseline: 2.8329x; 2.8329x over previous
import functools

import jax
import jax.numpy as jnp
from jax import lax
from jax.experimental import pallas as pl
from jax.experimental.pallas import tpu as pltpu
from jax.experimental.pallas import tpu_sc as plsc

DIM = 128
MARGIN = 4.0
BATCH = 4096
NC = 2
NS = 16
NW = NC * NS
L = 16
TRIPLES_PER_W = BATCH // NW
GROUPS = TRIPLES_PER_W // L


def _rsqrt(x):
    x = jnp.maximum(x, 1e-30)
    i = plsc.bitcast(x, jnp.int32)
    i = 0x5F3759DF - (i >> 1)
    y = plsc.bitcast(i, jnp.float32)
    for _ in range(3):
        y = y * (1.5 - 0.5 * x * y * y)
    return y


def _distance(bufs, lanes):
    hb, rb, tb, htb, rtb, ttb = bufs
    zeros = jnp.zeros((L,), jnp.float32)

    def body(d, c):
        dcol = (jnp.broadcast_to(d, (L,)).astype(jnp.int32) + lanes) & (DIM - 1)
        h = plsc.load_gather(hb, [lanes, dcol])
        r = plsc.load_gather(rb, [lanes, dcol])
        t = plsc.load_gather(tb, [lanes, dcol])
        ht = plsc.load_gather(htb, [lanes, dcol])
        rt = plsc.load_gather(rtb, [lanes, dcol])
        tt = plsc.load_gather(ttb, [lanes, dcol])
        return (
            c[0] + h * ht,
            c[1] + t * tt,
            c[2] + h * r,
            c[3] + h * t,
            c[4] + r * t,
            c[5] + h * rt,
            c[6] + t * rt,
            c[7] + r * rt,
            c[8] + rt * rt,
            c[9] + r * r,
            c[10] + h * h,
            c[11] + t * t,
        )

    (sh, st, hr, ht_d, rt_d, hrt, trt, rrt, rtrt, rr, hh, tt2) = lax.fori_loop(
        0, DIM, body, (zeros,) * 12)

    nh2 = hh + 2.0 * sh * hrt + sh * sh * rtrt
    nt2 = tt2 + 2.0 * st * trt + st * st * rtrt
    hp_r = hr + sh * rrt
    hp_tp = ht_d + st * hrt + sh * trt + sh * st * rtrt
    r_tp = rt_d + st * rrt
    inh = _rsqrt(nh2)
    int_ = _rsqrt(nt2)
    inr = _rsqrt(rr)
    d2 = 3.0 + 2.0 * (hp_r * inh * inr - hp_tp * inh * int_ - r_tp * inr * int_)
    d2 = jnp.maximum(d2, 0.0)
    return d2 * _rsqrt(d2)


def _make_sc_kernel():
    mesh = plsc.VectorSubcoreMesh(core_axis_name="c", subcore_axis_name="s")

    @functools.partial(
        pl.kernel,
        mesh=mesh,
        compiler_params=pltpu.CompilerParams(needs_layout_passes=False),
        out_type=jax.ShapeDtypeStruct((NW, L), jnp.float32),
        scratch_types=(
            [pltpu.VMEM((6, TRIPLES_PER_W), jnp.int32)]
            + [pltpu.VMEM((L, DIM), jnp.float32) for _ in range(12)]
            + [pltpu.VMEM((L,), jnp.float32), pltpu.SemaphoreType.DMA]
        ),
    )
    def sc_kernel(ent_emb, rel_emb, ent_tr, rel_tr, idx_hbm, out_hbm,
                  idx_v, b0, b1, b2, b3, b4, b5, b6, b7, b8, b9, b10, b11,
                  acc_v, sem):
        wid = lax.axis_index("s") * NC + lax.axis_index("c")
        base = wid * TRIPLES_PER_W
        for j in range(6):
            pltpu.sync_copy(idx_hbm.at[j, pl.ds(base, TRIPLES_PER_W)],
                            idx_v.at[j])

        lanes = lax.iota(jnp.int32, L)
        pos_bufs = (b0, b1, b2, b3, b4, b5)
        neg_bufs = (b6, b7, b8, b9, b10, b11)
        plan = [(ent_emb, 0), (rel_emb, 1), (ent_emb, 2),
                (ent_tr, 0), (rel_tr, 1), (ent_tr, 2),
                (ent_emb, 3), (rel_emb, 4), (ent_emb, 5),
                (ent_tr, 3), (rel_tr, 4), (ent_tr, 5)]

        def group(g, acc):
            off = g * L
            copies = [
                pltpu.async_copy(tbl.at[idx_v.at[row, pl.ds(off, L)]], buf, sem)
                for buf, (tbl, row) in zip(pos_bufs + neg_bufs, plan)
            ]
            for c in copies:
                c.wait()
            pos = _distance(pos_bufs, lanes)
            neg = _distance(neg_bufs, lanes)
            return acc + jnp.maximum(pos - neg + MARGIN, 0.0)

        acc = lax.fori_loop(0, GROUPS, group, jnp.zeros((L,), jnp.float32))
        acc_v[...] = acc
        pltpu.sync_copy(acc_v, out_hbm.at[wid])

    return sc_kernel


_SC_KERNEL = _make_sc_kernel()


@jax.jit
def kernel(current_triples, corrupted_triples, ent_embedding, rel_embedding,
           ent_transfer, rel_transfer):
    idx = jnp.concatenate(
        [current_triples.T, corrupted_triples.T], axis=0).astype(jnp.int32)
    partials = _SC_KERNEL(ent_embedding, rel_embedding, ent_transfer,
                          rel_transfer, idx)
    return jnp.sum(partials) / BATCH


# speedup vs baseline: 3.5392x; 1.2493x over previous
import functools

import jax
import jax.numpy as jnp
from jax import lax
from jax.experimental import pallas as pl
from jax.experimental.pallas import tpu as pltpu
from jax.experimental.pallas import tpu_sc as plsc

DIM = 128
MARGIN = 4.0
BATCH = 4096
NC = 2
NS = 16
NW = NC * NS
L = 16
TRIPLES_PER_W = BATCH // NW
GROUPS = TRIPLES_PER_W // L


def _rsqrt(x):
    x = jnp.maximum(x, 1e-30)
    i = plsc.bitcast(x, jnp.int32)
    i = 0x5F3759DF - (i >> 1)
    y = plsc.bitcast(i, jnp.float32)
    for _ in range(3):
        y = y * (1.5 - 0.5 * x * y * y)
    return y


def _distance(ebuf, etbuf, rbuf, rtbuf, lanes, ho, to, ro):
    zeros = jnp.zeros((L,), jnp.float32)
    hrow = lanes + ho
    trow = lanes + to
    rrow = lanes + ro

    def body(d, c):
        dcol = (jnp.broadcast_to(d, (L,)).astype(jnp.int32) + lanes) & (DIM - 1)
        h = plsc.load_gather(ebuf, [hrow, dcol])
        t = plsc.load_gather(ebuf, [trow, dcol])
        ht = plsc.load_gather(etbuf, [hrow, dcol])
        tt = plsc.load_gather(etbuf, [trow, dcol])
        r = plsc.load_gather(rbuf, [rrow, dcol])
        rt = plsc.load_gather(rtbuf, [rrow, dcol])
        return (
            c[0] + h * ht,
            c[1] + t * tt,
            c[2] + h * r,
            c[3] + h * t,
            c[4] + r * t,
            c[5] + h * rt,
            c[6] + t * rt,
            c[7] + r * rt,
            c[8] + rt * rt,
            c[9] + r * r,
            c[10] + h * h,
            c[11] + t * t,
        )

    (sh, st, hr, ht_d, rt_d, hrt, trt, rrt, rtrt, rr, hh, tt2) = lax.fori_loop(
        0, DIM, body, (zeros,) * 12)

    nh2 = hh + 2.0 * sh * hrt + sh * sh * rtrt
    nt2 = tt2 + 2.0 * st * trt + st * st * rtrt
    hp_r = hr + sh * rrt
    hp_tp = ht_d + st * hrt + sh * trt + sh * st * rtrt
    r_tp = rt_d + st * rrt
    inh = _rsqrt(nh2)
    int_ = _rsqrt(nt2)
    inr = _rsqrt(rr)
    d2 = 3.0 + 2.0 * (hp_r * inh * inr - hp_tp * inh * int_ - r_tp * inr * int_)
    d2 = jnp.maximum(d2, 0.0)
    return d2 * _rsqrt(d2)


def _make_sc_kernel():
    mesh = plsc.VectorSubcoreMesh(core_axis_name="c", subcore_axis_name="s")

    @functools.partial(
        pl.kernel,
        mesh=mesh,
        compiler_params=pltpu.CompilerParams(needs_layout_passes=False),
        out_type=jax.ShapeDtypeStruct((NW, L), jnp.float32),
        scratch_types=(
            [pltpu.VMEM((GROUPS, 4 * L), jnp.int32),
             pltpu.VMEM((GROUPS, 2 * L), jnp.int32)]
            + [pltpu.VMEM((4 * L, DIM), jnp.float32) for _ in range(4)]
            + [pltpu.VMEM((2 * L, DIM), jnp.float32) for _ in range(4)]
            + [pltpu.VMEM((L,), jnp.float32),
               pltpu.SemaphoreType.DMA, pltpu.SemaphoreType.DMA]
        ),
    )
    def sc_kernel(ent_emb, rel_emb, ent_tr, rel_tr, eidx_hbm, ridx_hbm,
                  out_hbm, eidx_v, ridx_v, ebA, ebB, etA, etB,
                  rbA, rbB, rtA, rtB, acc_v, semA, semB):
        wid = lax.axis_index("s") * NC + lax.axis_index("c")
        pltpu.sync_copy(eidx_hbm.at[wid], eidx_v)
        pltpu.sync_copy(ridx_hbm.at[wid], ridx_v)
        lanes = lax.iota(jnp.int32, L)
        sets = ((ebA, etA, rbA, rtA, semA), (ebB, etB, rbB, rtB, semB))

        def fire(g, s):
            eb, et, rb, rt, sem = s
            pltpu.async_copy(ent_emb.at[eidx_v.at[g]], eb, sem)
            pltpu.async_copy(ent_tr.at[eidx_v.at[g]], et, sem)
            pltpu.async_copy(rel_emb.at[ridx_v.at[g]], rb, sem)
            pltpu.async_copy(rel_tr.at[ridx_v.at[g]], rt, sem)

        def drain(s):
            eb, et, rb, rt, sem = s
            pltpu.make_async_copy(ent_emb.at[eidx_v.at[0]], eb, sem).wait()
            pltpu.make_async_copy(ent_tr.at[eidx_v.at[0]], et, sem).wait()
            pltpu.make_async_copy(rel_emb.at[ridx_v.at[0]], rb, sem).wait()
            pltpu.make_async_copy(rel_tr.at[ridx_v.at[0]], rt, sem).wait()

        def compute(s, acc):
            eb, et, rb, rt, _ = s
            pos = _distance(eb, et, rb, rt, lanes, 0, L, 0)
            neg = _distance(eb, et, rb, rt, lanes, 2 * L, 3 * L, L)
            return acc + jnp.maximum(pos - neg + MARGIN, 0.0)

        fire(0, sets[0])

        def pair(gg, acc):
            fire(2 * gg + 1, sets[1])
            drain(sets[0])
            acc = compute(sets[0], acc)

            @pl.when(gg < GROUPS // 2 - 1)
            def _():
                fire(2 * gg + 2, sets[0])

            drain(sets[1])
            return compute(sets[1], acc)

        acc = lax.fori_loop(0, GROUPS // 2, pair, jnp.zeros((L,), jnp.float32))
        acc_v[...] = acc
        pltpu.sync_copy(acc_v, out_hbm.at[wid])

    return sc_kernel


_SC_KERNEL = _make_sc_kernel()


@jax.jit
def kernel(current_triples, corrupted_triples, ent_embedding, rel_embedding,
           ent_transfer, rel_transfer):
    cur = current_triples.astype(jnp.int32)
    cor = corrupted_triples.astype(jnp.int32)
    def wg(col_arrays):
        parts = [a.reshape(NW, GROUPS, L) for a in col_arrays]
        return jnp.stack(parts, axis=2).reshape(NW, GROUPS, len(parts) * L)

    eidx = wg([cur[:, 0], cur[:, 2], cor[:, 0], cor[:, 2]])
    ridx = wg([cur[:, 1], cor[:, 1]])
    partials = _SC_KERNEL(ent_embedding, rel_embedding, ent_transfer,
                          rel_transfer, eidx, ridx)
    return jnp.sum(partials) / BATCH
